# baseline (speedup 1.0000x reference)
import functools

import numpy as np
import jax
import jax.numpy as jnp
from jax import lax
from jax.experimental import pallas as pl
from jax.experimental.pallas import tpu as pltpu
from jax.experimental.pallas import tpu_sc as plsc

N = 10000
E = 320000
G = 64
D_IN = 128
D_H = 256
EPS = 1e-5

NC = 2
NS = 16
NPAD = 10240
EPAD = 327680
ROWS = EPAD // 128
TILE_ROWS = NPAD // NS

_BLK = 256
_GRID = NPAD // _BLK

def _mesh():
    return plsc.VectorSubcoreMesh(core_axis_name="c", subcore_axis_name="s",
                                  num_cores=NC, num_subcores=NS)


def _zero_rows(rows_v):
    zv = jnp.zeros((16,), jnp.float32)

    def zrow(r, _):
        def zcol(j, __):
            rows_v[r, pl.ds(j * 16, 16)] = zv
            return 0
        return lax.fori_loop(0, 8, zcol, 0)
    lax.fori_loop(0, 128, zrow, 0)


def _sc_agg1(x_pad, src2d, dst2d):
    nchunk = ROWS // (NC * NS)

    @functools.partial(
        pl.kernel,
        out_type=[
            jax.ShapeDtypeStruct((NC * NPAD, 128), jnp.float32),
            jax.ShapeDtypeStruct((NC * NPAD,), jnp.float32),
        ],
        mesh=_mesh(),
        scratch_types=[
            pltpu.VMEM_SHARED((NPAD, 128), jnp.float32),
            pltpu.VMEM_SHARED((NPAD,), jnp.float32),
            pltpu.VMEM((16, 128), jnp.int32),
            pltpu.VMEM((16, 128), jnp.int32),
            pltpu.VMEM((128, 128), jnp.float32),
            pltpu.VMEM((128, 128), jnp.float32),
            pltpu.VMEM((128,), jnp.float32),
            pltpu.SemaphoreType.DMA,
            pltpu.SemaphoreType.DMA,
            pltpu.SemaphoreType.DMA,
            pltpu.SemaphoreType.DMA,
            pltpu.SemaphoreType.DMA,
        ],
    )
    def k(x_hbm, src_hbm, dst_hbm, out_hbm, deg_hbm,
          acc_sp, deg_sp, si_v, di_v, rows_a, rows_b, ones_v,
          sem_ga, sem_gb, sem_sa, sem_sb, sem_d):
        c = lax.axis_index("c")
        s = lax.axis_index("s")
        wid = c * NS + s
        _zero_rows(rows_a)
        _zero_rows(rows_b)
        one = jnp.ones((16,), jnp.float32)
        for j in range(8):
            ones_v[pl.ds(j * 16, 16)] = one

        def zero_step(kk, _):
            pltpu.sync_copy(rows_a, acc_sp.at[pl.ds(s * TILE_ROWS + kk * 128, 128)])
            pltpu.sync_copy(rows_a.at[0], deg_sp.at[pl.ds(s * TILE_ROWS + kk * 128, 128)])
            return 0
        lax.fori_loop(0, TILE_ROWS // 128, zero_step, 0)

        pltpu.sync_copy(dst_hbm.at[pl.ds(wid * nchunk, 16)], di_v)
        plsc.subcore_barrier()
        pltpu.async_copy(rows_b, acc_sp.at[di_v.at[15]], sem_sb, add=True)
        pltpu.async_copy(rows_b.at[0], deg_sp.at[di_v.at[15]], sem_d, add=True)

        def group(g, _):
            pltpu.make_async_copy(rows_b, acc_sp.at[di_v.at[15]], sem_sb).wait()
            pltpu.make_async_copy(ones_v, deg_sp.at[di_v.at[15]], sem_d).wait()
            base = wid * nchunk + g * 16
            pltpu.sync_copy(src_hbm.at[pl.ds(base, 16)], si_v)
            pltpu.sync_copy(dst_hbm.at[pl.ds(base, 16)], di_v)
            pltpu.async_copy(x_hbm.at[si_v.at[0]], rows_a, sem_ga)

            pltpu.make_async_copy(x_hbm.at[si_v.at[0]], rows_a, sem_ga).wait()
            pltpu.async_copy(x_hbm.at[si_v.at[1]], rows_b, sem_gb)
            pltpu.async_copy(rows_a, acc_sp.at[di_v.at[0]], sem_sa, add=True)
            pltpu.async_copy(ones_v, deg_sp.at[di_v.at[0]], sem_d, add=True)
            pltpu.make_async_copy(x_hbm.at[si_v.at[1]], rows_b, sem_gb).wait()
            pltpu.make_async_copy(rows_a, acc_sp.at[di_v.at[0]], sem_sa).wait()
            pltpu.make_async_copy(ones_v, deg_sp.at[di_v.at[0]], sem_d).wait()
            pltpu.async_copy(x_hbm.at[si_v.at[2]], rows_a, sem_ga)
            pltpu.async_copy(rows_b, acc_sp.at[di_v.at[1]], sem_sb, add=True)
            pltpu.async_copy(ones_v, deg_sp.at[di_v.at[1]], sem_d, add=True)

            def pair(p, __):
                j = 2 * p
                pltpu.make_async_copy(x_hbm.at[si_v.at[j]], rows_a, sem_ga).wait()
                pltpu.make_async_copy(rows_b, acc_sp.at[di_v.at[j - 1]], sem_sb).wait()
                pltpu.make_async_copy(ones_v, deg_sp.at[di_v.at[j - 1]], sem_d).wait()
                pltpu.async_copy(x_hbm.at[si_v.at[j + 1]], rows_b, sem_gb)
                pltpu.async_copy(rows_a, acc_sp.at[di_v.at[j]], sem_sa, add=True)
                pltpu.async_copy(ones_v, deg_sp.at[di_v.at[j]], sem_d, add=True)
                pltpu.make_async_copy(x_hbm.at[si_v.at[j + 1]], rows_b, sem_gb).wait()
                pltpu.make_async_copy(rows_a, acc_sp.at[di_v.at[j]], sem_sa).wait()
                pltpu.make_async_copy(ones_v, deg_sp.at[di_v.at[j]], sem_d).wait()

                @pl.when(p < 7)
                def _():
                    pltpu.async_copy(x_hbm.at[si_v.at[j + 2]], rows_a, sem_ga)
                pltpu.async_copy(rows_b, acc_sp.at[di_v.at[j + 1]], sem_sb, add=True)
                pltpu.async_copy(ones_v, deg_sp.at[di_v.at[j + 1]], sem_d, add=True)
                return 0
            return lax.fori_loop(1, 8, pair, 0)
        lax.fori_loop(0, nchunk // 16, group, 0)
        pltpu.make_async_copy(rows_b, acc_sp.at[di_v.at[15]], sem_sb).wait()
        pltpu.make_async_copy(ones_v, deg_sp.at[di_v.at[15]], sem_d).wait()
        plsc.subcore_barrier()

        pltpu.sync_copy(
            acc_sp.at[pl.ds(s * TILE_ROWS, TILE_ROWS)],
            out_hbm.at[pl.ds(c * NPAD + s * TILE_ROWS, TILE_ROWS)])
        pltpu.sync_copy(
            deg_sp.at[pl.ds(s * TILE_ROWS, TILE_ROWS)],
            deg_hbm.at[pl.ds(c * NPAD + s * TILE_ROWS, TILE_ROWS)])

    return k(x_pad, src2d, dst2d)


def _sc_agg2(h0, h1, src2d, dst2d):
    nchunk = ROWS // NS

    @functools.partial(
        pl.kernel,
        out_type=jax.ShapeDtypeStruct((NC * NPAD, 128), jnp.float32),
        mesh=_mesh(),
        scratch_types=[
            pltpu.VMEM_SHARED((NPAD, 128), jnp.float32),
            pltpu.VMEM((16, 128), jnp.int32),
            pltpu.VMEM((16, 128), jnp.int32),
            pltpu.VMEM((128, 128), jnp.float32),
            pltpu.VMEM((128, 128), jnp.float32),
            pltpu.SemaphoreType.DMA,
            pltpu.SemaphoreType.DMA,
            pltpu.SemaphoreType.DMA,
            pltpu.SemaphoreType.DMA,
        ],
    )
    def k(h0_hbm, h1_hbm, src_hbm, dst_hbm, out_hbm,
          acc_sp, si_v, di_v, rows_a, rows_b,
          sem_ga, sem_gb, sem_sa, sem_sb):
        c = lax.axis_index("c")
        s = lax.axis_index("s")
        _zero_rows(rows_a)
        _zero_rows(rows_b)

        def zero_step(kk, _):
            pltpu.sync_copy(rows_a, acc_sp.at[pl.ds(s * TILE_ROWS + kk * 128, 128)])
            return 0
        lax.fori_loop(0, TILE_ROWS // 128, zero_step, 0)

        pltpu.sync_copy(dst_hbm.at[pl.ds(s * nchunk, 16)], di_v)
        plsc.subcore_barrier()
        pltpu.async_copy(rows_b, acc_sp.at[di_v.at[15]], sem_sb, add=True)

        def make_group(tab):
            def group(g, _):
                pltpu.make_async_copy(rows_b, acc_sp.at[di_v.at[15]], sem_sb).wait()
                base = s * nchunk + g * 16
                pltpu.sync_copy(src_hbm.at[pl.ds(base, 16)], si_v)
                pltpu.sync_copy(dst_hbm.at[pl.ds(base, 16)], di_v)
                pltpu.async_copy(tab.at[si_v.at[0]], rows_a, sem_ga)

                pltpu.make_async_copy(tab.at[si_v.at[0]], rows_a, sem_ga).wait()
                pltpu.async_copy(tab.at[si_v.at[1]], rows_b, sem_gb)
                pltpu.async_copy(rows_a, acc_sp.at[di_v.at[0]], sem_sa, add=True)
                pltpu.make_async_copy(tab.at[si_v.at[1]], rows_b, sem_gb).wait()
                pltpu.make_async_copy(rows_a, acc_sp.at[di_v.at[0]], sem_sa).wait()
                pltpu.async_copy(tab.at[si_v.at[2]], rows_a, sem_ga)
                pltpu.async_copy(rows_b, acc_sp.at[di_v.at[1]], sem_sb, add=True)

                def pair(p, __):
                    j = 2 * p
                    pltpu.make_async_copy(tab.at[si_v.at[j]], rows_a, sem_ga).wait()
                    pltpu.make_async_copy(rows_b, acc_sp.at[di_v.at[j - 1]], sem_sb).wait()
                    pltpu.async_copy(tab.at[si_v.at[j + 1]], rows_b, sem_gb)
                    pltpu.async_copy(rows_a, acc_sp.at[di_v.at[j]], sem_sa, add=True)
                    pltpu.make_async_copy(tab.at[si_v.at[j + 1]], rows_b, sem_gb).wait()
                    pltpu.make_async_copy(rows_a, acc_sp.at[di_v.at[j]], sem_sa).wait()

                    @pl.when(p < 7)
                    def _():
                        pltpu.async_copy(tab.at[si_v.at[j + 2]], rows_a, sem_ga)
                    pltpu.async_copy(rows_b, acc_sp.at[di_v.at[j + 1]], sem_sb, add=True)
                    return 0
                return lax.fori_loop(1, 8, pair, 0)
            return group

        @pl.when(c == 0)
        def _():
            lax.fori_loop(0, nchunk // 16, make_group(h0_hbm), 0)

        @pl.when(c == 1)
        def _():
            lax.fori_loop(0, nchunk // 16, make_group(h1_hbm), 0)

        pltpu.make_async_copy(rows_b, acc_sp.at[di_v.at[15]], sem_sb).wait()
        plsc.subcore_barrier()
        pltpu.sync_copy(
            acc_sp.at[pl.ds(s * TILE_ROWS, TILE_ROWS)],
            out_hbm.at[pl.ds(c * NPAD + s * TILE_ROWS, TILE_ROWS)])

    return k(h0, h1, src2d, dst2d)


def _dotT(a, w):
    return lax.dot_general(a, w, (((1,), (1,)), ((), ())),
                           preferred_element_type=jnp.float32)


def _tc_layer1(agg1, deg1, x_pad, W1l, W1r, b1l, b1r, bn_g, bn_b):
    bnscale = float(1.0 / np.sqrt(1.0 + EPS))

    def body(agg_a_ref, agg_b_ref, deg_a_ref, deg_b_ref, x_ref,
             wl_ref, wr_ref, bl_ref, br_ref,
             g_ref, bb_ref, h0_ref, h1_ref, dinv_ref):
        agg = agg_a_ref[...] + agg_b_ref[...]
        deg = jnp.maximum(deg_a_ref[...] + deg_b_ref[...], 1.0)[:, None]
        mean = agg / deg
        xb = x_ref[...]
        acc = _dotT(mean, wl_ref[...]) + _dotT(xb, wr_ref[...])
        h = (acc + bl_ref[...] + br_ref[...]) * (g_ref[...] * bnscale) + bb_ref[...]
        h = jnp.maximum(h, 0.0)
        h0_ref[...] = h[:, :128]
        h1_ref[...] = h[:, 128:]
        dinv_ref[...] = 1.0 / deg

    return pl.pallas_call(
        body,
        grid=(_GRID,),
        in_specs=[
            pl.BlockSpec((_BLK, 128), lambda i: (i, 0)),
            pl.BlockSpec((_BLK, 128), lambda i: (i + _GRID, 0)),
            pl.BlockSpec((_BLK,), lambda i: (i,)),
            pl.BlockSpec((_BLK,), lambda i: (i + _GRID,)),
            pl.BlockSpec((_BLK, 128), lambda i: (i, 0)),
            pl.BlockSpec((D_H, D_IN), lambda i: (0, 0)),
            pl.BlockSpec((D_H, D_IN), lambda i: (0, 0)),
            pl.BlockSpec((D_H,), lambda i: (0,)),
            pl.BlockSpec((D_H,), lambda i: (0,)),
            pl.BlockSpec((D_H,), lambda i: (0,)),
            pl.BlockSpec((D_H,), lambda i: (0,)),
        ],
        out_specs=[
            pl.BlockSpec((_BLK, 128), lambda i: (i, 0)),
            pl.BlockSpec((_BLK, 128), lambda i: (i, 0)),
            pl.BlockSpec((_BLK, 1), lambda i: (i, 0)),
        ],
        out_shape=[
            jax.ShapeDtypeStruct((NPAD, 128), jnp.float32),
            jax.ShapeDtypeStruct((NPAD, 128), jnp.float32),
            jax.ShapeDtypeStruct((NPAD, 1), jnp.float32),
        ],
    )(agg1, agg1, deg1, deg1, x_pad, W1l, W1r, b1l, b1r, bn_g, bn_b)


def _tc_final(agg2, h0, h1, dinv, batch_p,
              W2l, b2l, W2r, b2r, Wc1, bc1, Wc2, bc2):
    def body(agg_a_ref, agg_b_ref, h0_ref, h1_ref, dinv_ref, b_ref,
             w2l_ref, b2l_ref, w2r_ref, b2r_ref,
             wc1_ref, bc1_ref, wc2_ref, bc2_ref,
             out_ref, accm, acch, accc):
        i = pl.program_id(0)

        @pl.when(i == 0)
        def _():
            accm[...] = jnp.zeros_like(accm)
            acch[...] = jnp.zeros_like(acch)
            accc[...] = jnp.zeros_like(accc)

        bvec = b_ref[...]
        onehot = (lax.broadcasted_iota(jnp.int32, (G, _BLK), 0)
                  == bvec[None, :]).astype(jnp.float32)
        m2 = jnp.concatenate([agg_a_ref[...], agg_b_ref[...]], axis=1) * dinv_ref[...]
        hb = jnp.concatenate([h0_ref[...], h1_ref[...]], axis=1)
        accm[...] += lax.dot_general(onehot, m2, (((1,), (0,)), ((), ())),
                                     preferred_element_type=jnp.float32)
        acch[...] += lax.dot_general(onehot, hb, (((1,), (0,)), ((), ())),
                                     preferred_element_type=jnp.float32)
        accc[...] += jnp.broadcast_to(
            jnp.sum(onehot, axis=1, keepdims=True), accc.shape)

        @pl.when(i == _GRID - 1)
        def _():
            cnt = accc[:, 0:1]
            ps = _dotT(accm[...], w2l_ref[...]) + _dotT(acch[...], w2r_ref[...])
            ps = ps + cnt * (b2l_ref[...] + b2r_ref[...])[None, :]
            pooled = ps / jnp.maximum(cnt, 1.0)
            z = jnp.maximum(_dotT(pooled, wc1_ref[...]) + bc1_ref[...][None, :], 0.0)
            logits = _dotT(z, wc2_ref[...]) + bc2_ref[...][None, :]
            mx = jnp.max(logits, axis=1, keepdims=True)
            lse = mx + jnp.log(jnp.sum(jnp.exp(logits - mx), axis=1, keepdims=True))
            out_ref[...] = logits - lse

    return pl.pallas_call(
        body,
        grid=(_GRID,),
        in_specs=[
            pl.BlockSpec((_BLK, 128), lambda i: (i, 0)),
            pl.BlockSpec((_BLK, 128), lambda i: (i + _GRID, 0)),
            pl.BlockSpec((_BLK, 128), lambda i: (i, 0)),
            pl.BlockSpec((_BLK, 128), lambda i: (i, 0)),
            pl.BlockSpec((_BLK, 1), lambda i: (i, 0)),
            pl.BlockSpec((_BLK,), lambda i: (i,)),
            pl.BlockSpec((D_H, D_H), lambda i: (0, 0)),
            pl.BlockSpec((D_H,), lambda i: (0,)),
            pl.BlockSpec((D_H, D_H), lambda i: (0, 0)),
            pl.BlockSpec((D_H,), lambda i: (0,)),
            pl.BlockSpec((D_H // 2, D_H), lambda i: (0, 0)),
            pl.BlockSpec((D_H // 2,), lambda i: (0,)),
            pl.BlockSpec((2, D_H // 2), lambda i: (0, 0)),
            pl.BlockSpec((2,), lambda i: (0,)),
        ],
        out_specs=pl.BlockSpec((G, 2), lambda i: (0, 0)),
        out_shape=jax.ShapeDtypeStruct((G, 2), jnp.float32),
        scratch_shapes=[
            pltpu.VMEM((G, D_H), jnp.float32),
            pltpu.VMEM((G, D_H), jnp.float32),
            pltpu.VMEM((G, 128), jnp.float32),
        ],
    )(agg2, agg2, h0, h1, dinv, batch_p,
      W2l, b2l, W2r, b2r, Wc1, bc1, Wc2, bc2)


def kernel(x, edge_index, batch, W1l, b1l, W1r, b1r, bn_g, bn_b,
           W2l, b2l, W2r, b2r, Wc1, bc1, Wc2, bc2):
    src = edge_index[0]
    dst = edge_index[1]
    pad_e = EPAD - E
    src2d = jnp.concatenate([src, jnp.zeros((pad_e,), jnp.int32)]).reshape(ROWS, 128)
    dst2d = jnp.concatenate([dst, jnp.full((pad_e,), NPAD - 1, jnp.int32)]).reshape(ROWS, 128)
    x_pad = jnp.concatenate(
        [x, jnp.zeros((NPAD - N, D_IN), jnp.float32)], axis=0)
    batch_p = jnp.concatenate([batch, jnp.full((NPAD - N,), G, jnp.int32)])

    agg1, deg1 = _sc_agg1(x_pad, src2d, dst2d)
    h0, h1, dinv = _tc_layer1(agg1, deg1, x_pad, W1l, W1r, b1l, b1r, bn_g, bn_b)
    agg2 = _sc_agg2(h0, h1, src2d, dst2d)
    return _tc_final(agg2, h0, h1, dinv, batch_p,
                     W2l, b2l, W2r, b2r, Wc1, bc1, Wc2, bc2)


# speedup vs baseline: 2.8611x; 2.8611x over previous
import functools

import numpy as np
import jax
import jax.numpy as jnp
from jax import lax
from jax.experimental import pallas as pl
from jax.experimental.pallas import tpu as pltpu
from jax.experimental.pallas import tpu_sc as plsc

N = 10000
E = 320000
G = 64
D_IN = 128
D_H = 256
EPS = 1e-5

NC = 2
NS = 16
NPAD = 10240
EPAD = 327680
ROWS = EPAD // 128
TILE_ROWS = NPAD // NS

_BLK = 1024
_GRID = NPAD // _BLK

def _mesh():
    return plsc.VectorSubcoreMesh(core_axis_name="c", subcore_axis_name="s",
                                  num_cores=NC, num_subcores=NS)


def _zero_rows(rows_v):
    zv = jnp.zeros((16,), jnp.float32)

    def zrow(r, _):
        def zcol(j, __):
            rows_v[r, pl.ds(j * 16, 16)] = zv
            return 0
        return lax.fori_loop(0, 8, zcol, 0)
    lax.fori_loop(0, 128, zrow, 0)


def _sc_agg1(x_pad, src2d, dst2d):
    C0, C1 = 80, 80
    NG0, NG1 = C0 // 16, C1 // 16

    @functools.partial(
        pl.kernel,
        out_type=[
            jax.ShapeDtypeStruct((NC * NPAD, 128), jnp.float32),
            jax.ShapeDtypeStruct((NC * NPAD,), jnp.float32),
        ],
        mesh=_mesh(),
        scratch_types=[
            pltpu.VMEM_SHARED((NPAD, 128), jnp.float32),
            pltpu.VMEM_SHARED((NPAD,), jnp.float32),
            pltpu.VMEM((16, 128), jnp.int32),
            pltpu.VMEM((16, 128), jnp.int32),
            pltpu.VMEM((128, 128), jnp.float32),
            pltpu.VMEM((128, 128), jnp.float32),
            pltpu.VMEM((128,), jnp.float32),
            pltpu.SemaphoreType.DMA,
            pltpu.SemaphoreType.DMA,
            pltpu.SemaphoreType.DMA,
        ],
    )
    def k(x_hbm, src_hbm, dst_hbm, out_hbm, deg_hbm,
          acc_sp, deg_sp, si_v, di_v, rows_a, rows_b, ones_v,
          sem_ga, sem_gb, sem_d):
        c = lax.axis_index("c")
        s = lax.axis_index("s")
        wid = c * NS + s
        _zero_rows(rows_a)
        one = jnp.ones((16,), jnp.float32)
        for j in range(8):
            ones_v[pl.ds(j * 16, 16)] = one

        def zero_step(kk, _):
            pltpu.sync_copy(rows_a, acc_sp.at[pl.ds(s * TILE_ROWS + kk * 128, 128)])
            pltpu.sync_copy(rows_a.at[0], deg_sp.at[pl.ds(s * TILE_ROWS + kk * 128, 128)])
            return 0
        lax.fori_loop(0, TILE_ROWS // 128, zero_step, 0)
        plsc.subcore_barrier()

        tile_base = jnp.where(c == 0, s * C0, NS * C0 + s * C1)
        ngroups = jnp.where(c == 0, NG0, NG1)

        def group(g, _):
            base = tile_base + g * 16
            pltpu.sync_copy(src_hbm.at[pl.ds(base, 16)], si_v)
            pltpu.sync_copy(dst_hbm.at[pl.ds(base, 16)], di_v)
            pltpu.async_copy(x_hbm.at[si_v.at[0]], rows_a, sem_ga)

            def pair(p, __):
                j = 2 * p
                pltpu.async_copy(x_hbm.at[si_v.at[j + 1]], rows_b, sem_gb)
                pltpu.make_async_copy(x_hbm.at[si_v.at[j]], rows_a, sem_ga).wait()
                pltpu.sync_copy(rows_a, acc_sp.at[di_v.at[j]], add=True)
                pltpu.async_copy(ones_v, deg_sp.at[di_v.at[j]], sem_d, add=True)

                @pl.when(p < 7)
                def _():
                    pltpu.async_copy(x_hbm.at[si_v.at[j + 2]], rows_a, sem_ga)
                pltpu.make_async_copy(x_hbm.at[si_v.at[j + 1]], rows_b, sem_gb).wait()
                pltpu.sync_copy(rows_b, acc_sp.at[di_v.at[j + 1]], add=True)
                pltpu.async_copy(ones_v, deg_sp.at[di_v.at[j + 1]], sem_d, add=True)
                return 0
            lax.fori_loop(0, 8, pair, 0)
            def ddrain(t, __):
                pltpu.make_async_copy(ones_v, deg_sp.at[di_v.at[0]], sem_d).wait()
                return 0
            return lax.fori_loop(0, 16, ddrain, 0)
        lax.fori_loop(0, ngroups, group, 0)
        plsc.subcore_barrier()

        pltpu.sync_copy(
            acc_sp.at[pl.ds(s * TILE_ROWS, TILE_ROWS)],
            out_hbm.at[pl.ds(c * NPAD + s * TILE_ROWS, TILE_ROWS)])
        pltpu.sync_copy(
            deg_sp.at[pl.ds(s * TILE_ROWS, TILE_ROWS)],
            deg_hbm.at[pl.ds(c * NPAD + s * TILE_ROWS, TILE_ROWS)])

    return k(x_pad, src2d, dst2d)


def _sc_agg2(h0, h1, src2d, dst2d):
    nchunk = ROWS // NS

    @functools.partial(
        pl.kernel,
        out_type=jax.ShapeDtypeStruct((NC * NPAD, 128), jnp.float32),
        mesh=_mesh(),
        scratch_types=[
            pltpu.VMEM_SHARED((NPAD, 128), jnp.float32),
            pltpu.VMEM((16, 128), jnp.int32),
            pltpu.VMEM((16, 128), jnp.int32),
            pltpu.VMEM((128, 128), jnp.float32),
            pltpu.VMEM((128, 128), jnp.float32),
            pltpu.SemaphoreType.DMA,
            pltpu.SemaphoreType.DMA,
            pltpu.SemaphoreType.DMA,
            pltpu.SemaphoreType.DMA,
        ],
    )
    def k(h0_hbm, h1_hbm, src_hbm, dst_hbm, out_hbm,
          acc_sp, si_v, di_v, rows_a, rows_b,
          sem_ga, sem_gb, sem_sa, sem_sb):
        c = lax.axis_index("c")
        s = lax.axis_index("s")
        _zero_rows(rows_a)

        def zero_step(kk, _):
            pltpu.sync_copy(rows_a, acc_sp.at[pl.ds(s * TILE_ROWS + kk * 128, 128)])
            return 0
        lax.fori_loop(0, TILE_ROWS // 128, zero_step, 0)
        plsc.subcore_barrier()

        def make_group(tab):
            def group(g, _):
                base = s * nchunk + g * 16
                pltpu.sync_copy(src_hbm.at[pl.ds(base, 16)], si_v)
                pltpu.sync_copy(dst_hbm.at[pl.ds(base, 16)], di_v)
                pltpu.async_copy(tab.at[si_v.at[0]], rows_a, sem_ga)

                def pair(p, __):
                    j = 2 * p
                    pltpu.async_copy(tab.at[si_v.at[j + 1]], rows_b, sem_gb)
                    pltpu.make_async_copy(tab.at[si_v.at[j]], rows_a, sem_ga).wait()
                    pltpu.sync_copy(rows_a, acc_sp.at[di_v.at[j]], add=True)

                    @pl.when(p < 7)
                    def _():
                        pltpu.async_copy(tab.at[si_v.at[j + 2]], rows_a, sem_ga)
                    pltpu.make_async_copy(tab.at[si_v.at[j + 1]], rows_b, sem_gb).wait()
                    pltpu.sync_copy(rows_b, acc_sp.at[di_v.at[j + 1]], add=True)
                    return 0
                return lax.fori_loop(0, 8, pair, 0)
            return group

        @pl.when(c == 0)
        def _():
            lax.fori_loop(0, nchunk // 16, make_group(h0_hbm), 0)

        @pl.when(c == 1)
        def _():
            lax.fori_loop(0, nchunk // 16, make_group(h1_hbm), 0)

        plsc.subcore_barrier()
        pltpu.sync_copy(
            acc_sp.at[pl.ds(s * TILE_ROWS, TILE_ROWS)],
            out_hbm.at[pl.ds(c * NPAD + s * TILE_ROWS, TILE_ROWS)])

    return k(h0, h1, src2d, dst2d)


def _dotT(a, w):
    return lax.dot_general(a, w, (((1,), (1,)), ((), ())),
                           preferred_element_type=jnp.float32)


def _tc_layer1(agg1, deg1, x_pad, W1l, W1r, b1l, b1r, bn_g, bn_b):
    bnscale = float(1.0 / np.sqrt(1.0 + EPS))

    def body(agg_a_ref, agg_b_ref, deg_a_ref, deg_b_ref, x_ref,
             wl_ref, wr_ref, bl_ref, br_ref,
             g_ref, bb_ref, h0_ref, h1_ref, dinv_ref):
        agg = agg_a_ref[...] + agg_b_ref[...]
        deg = jnp.maximum(deg_a_ref[...] + deg_b_ref[...], 1.0)[:, None]
        mean = agg / deg
        xb = x_ref[...]
        acc = _dotT(mean, wl_ref[...]) + _dotT(xb, wr_ref[...])
        h = (acc + bl_ref[...] + br_ref[...]) * (g_ref[...] * bnscale) + bb_ref[...]
        h = jnp.maximum(h, 0.0)
        h0_ref[...] = h[:, :128]
        h1_ref[...] = h[:, 128:]
        dinv_ref[...] = 1.0 / deg

    return pl.pallas_call(
        body,
        grid=(_GRID,),
        in_specs=[
            pl.BlockSpec((_BLK, 128), lambda i: (i, 0)),
            pl.BlockSpec((_BLK, 128), lambda i: (i + _GRID, 0)),
            pl.BlockSpec((_BLK,), lambda i: (i,)),
            pl.BlockSpec((_BLK,), lambda i: (i + _GRID,)),
            pl.BlockSpec((_BLK, 128), lambda i: (i, 0)),
            pl.BlockSpec((D_H, D_IN), lambda i: (0, 0)),
            pl.BlockSpec((D_H, D_IN), lambda i: (0, 0)),
            pl.BlockSpec((D_H,), lambda i: (0,)),
            pl.BlockSpec((D_H,), lambda i: (0,)),
            pl.BlockSpec((D_H,), lambda i: (0,)),
            pl.BlockSpec((D_H,), lambda i: (0,)),
        ],
        out_specs=[
            pl.BlockSpec((_BLK, 128), lambda i: (i, 0)),
            pl.BlockSpec((_BLK, 128), lambda i: (i, 0)),
            pl.BlockSpec((_BLK, 1), lambda i: (i, 0)),
        ],
        out_shape=[
            jax.ShapeDtypeStruct((NPAD, 128), jnp.float32),
            jax.ShapeDtypeStruct((NPAD, 128), jnp.float32),
            jax.ShapeDtypeStruct((NPAD, 1), jnp.float32),
        ],
    )(agg1, agg1, deg1, deg1, x_pad, W1l, W1r, b1l, b1r, bn_g, bn_b)


def _tc_final(agg2, h0, h1, dinv, batch_p,
              W2l, b2l, W2r, b2r, Wc1, bc1, Wc2, bc2):
    def body(agg_a_ref, agg_b_ref, h0_ref, h1_ref, dinv_ref, b_ref,
             w2l_ref, b2l_ref, w2r_ref, b2r_ref,
             wc1_ref, bc1_ref, wc2_ref, bc2_ref,
             out_ref, accm, acch, accc):
        i = pl.program_id(0)

        @pl.when(i == 0)
        def _():
            accm[...] = jnp.zeros_like(accm)
            acch[...] = jnp.zeros_like(acch)
            accc[...] = jnp.zeros_like(accc)

        bvec = b_ref[...]
        onehot = (lax.broadcasted_iota(jnp.int32, (G, _BLK), 0)
                  == bvec[None, :]).astype(jnp.float32)
        m2 = jnp.concatenate([agg_a_ref[...], agg_b_ref[...]], axis=1) * dinv_ref[...]
        hb = jnp.concatenate([h0_ref[...], h1_ref[...]], axis=1)
        accm[...] += lax.dot_general(onehot, m2, (((1,), (0,)), ((), ())),
                                     preferred_element_type=jnp.float32)
        acch[...] += lax.dot_general(onehot, hb, (((1,), (0,)), ((), ())),
                                     preferred_element_type=jnp.float32)
        accc[...] += jnp.broadcast_to(
            jnp.sum(onehot, axis=1, keepdims=True), accc.shape)

        @pl.when(i == _GRID - 1)
        def _():
            cnt = accc[:, 0:1]
            ps = _dotT(accm[...], w2l_ref[...]) + _dotT(acch[...], w2r_ref[...])
            ps = ps + cnt * (b2l_ref[...] + b2r_ref[...])[None, :]
            pooled = ps / jnp.maximum(cnt, 1.0)
            z = jnp.maximum(_dotT(pooled, wc1_ref[...]) + bc1_ref[...][None, :], 0.0)
            logits = _dotT(z, wc2_ref[...]) + bc2_ref[...][None, :]
            mx = jnp.max(logits, axis=1, keepdims=True)
            lse = mx + jnp.log(jnp.sum(jnp.exp(logits - mx), axis=1, keepdims=True))
            out_ref[...] = logits - lse

    return pl.pallas_call(
        body,
        grid=(_GRID,),
        in_specs=[
            pl.BlockSpec((_BLK, 128), lambda i: (i, 0)),
            pl.BlockSpec((_BLK, 128), lambda i: (i + _GRID, 0)),
            pl.BlockSpec((_BLK, 128), lambda i: (i, 0)),
            pl.BlockSpec((_BLK, 128), lambda i: (i, 0)),
            pl.BlockSpec((_BLK, 1), lambda i: (i, 0)),
            pl.BlockSpec((_BLK,), lambda i: (i,)),
            pl.BlockSpec((D_H, D_H), lambda i: (0, 0)),
            pl.BlockSpec((D_H,), lambda i: (0,)),
            pl.BlockSpec((D_H, D_H), lambda i: (0, 0)),
            pl.BlockSpec((D_H,), lambda i: (0,)),
            pl.BlockSpec((D_H // 2, D_H), lambda i: (0, 0)),
            pl.BlockSpec((D_H // 2,), lambda i: (0,)),
            pl.BlockSpec((2, D_H // 2), lambda i: (0, 0)),
            pl.BlockSpec((2,), lambda i: (0,)),
        ],
        out_specs=pl.BlockSpec((G, 2), lambda i: (0, 0)),
        out_shape=jax.ShapeDtypeStruct((G, 2), jnp.float32),
        scratch_shapes=[
            pltpu.VMEM((G, D_H), jnp.float32),
            pltpu.VMEM((G, D_H), jnp.float32),
            pltpu.VMEM((G, 128), jnp.float32),
        ],
    )(agg2, agg2, h0, h1, dinv, batch_p,
      W2l, b2l, W2r, b2r, Wc1, bc1, Wc2, bc2)


def kernel(x, edge_index, batch, W1l, b1l, W1r, b1r, bn_g, bn_b,
           W2l, b2l, W2r, b2r, Wc1, bc1, Wc2, bc2):
    src = edge_index[0]
    dst = edge_index[1]
    pad_e = EPAD - E
    pad_idx = jnp.arange(pad_e, dtype=jnp.int32)
    pad_src = pad_idx % N
    pad_dst = N + (pad_idx % (NPAD - N))
    src2d = jnp.concatenate([src, pad_src]).reshape(ROWS, 128)
    dst2d = jnp.concatenate([dst, pad_dst]).reshape(ROWS, 128)
    x_pad = jnp.concatenate(
        [x, jnp.zeros((NPAD - N, D_IN), jnp.float32)], axis=0)
    batch_p = jnp.concatenate([batch, jnp.full((NPAD - N,), G, jnp.int32)])

    agg1, deg1 = _sc_agg1(x_pad, src2d, dst2d)
    h0, h1, dinv = _tc_layer1(agg1, deg1, x_pad, W1l, W1r, b1l, b1r, bn_g, bn_b)
    agg2 = _sc_agg2(h0, h1, src2d, dst2d)
    return _tc_final(agg2, h0, h1, dinv, batch_p,
                     W2l, b2l, W2r, b2r, Wc1, bc1, Wc2, bc2)
